# baseline (speedup 1.0000x reference)
import functools

import jax
import jax.numpy as jnp
from jax import lax
from jax.experimental import pallas as pl
from jax.experimental.pallas import tpu as pltpu
from jax.experimental.pallas import tpu_sc as plsc

BATCH = 4096
HIST = 200
VOCAB = 100000
DIM = 16
OUT = 8
N = BATCH * HIST

NUM_CORES = 2
NUM_SUBCORES = 16
NUM_WORKERS = NUM_CORES * NUM_SUBCORES
BAGS_PER_W = BATCH // NUM_WORKERS
IDX_PER_W = BAGS_PER_W * HIST
OUT_PER_W = BAGS_PER_W * OUT

PAIR_IDX = 2 * HIST
CHUNKS = (128, 128, 128, 16)
PAIRS_PER_W = BAGS_PER_W // 2
NBUF = 4


def _tc_project(emb_table, w_scaled):
    def proj_kernel(t_ref, w_ref, o_ref):
        o_ref[...] = jnp.dot(t_ref[...], w_ref[...],
                             preferred_element_type=jnp.float32)

    blk = VOCAB // 10
    return pl.pallas_call(
        proj_kernel,
        grid=(10,),
        in_specs=[
            pl.BlockSpec((blk, DIM), lambda i: (i, 0)),
            pl.BlockSpec((DIM, OUT), lambda i: (0, 0)),
        ],
        out_specs=pl.BlockSpec((blk, OUT), lambda i: (i, 0)),
        out_shape=jax.ShapeDtypeStruct((VOCAB, OUT), jnp.float32),
    )(emb_table, w_scaled)


def _sc_bag_sum_kernel():
    mesh = plsc.VectorSubcoreMesh(core_axis_name="c", subcore_axis_name="s")

    @functools.partial(
        pl.kernel,
        mesh=mesh,
        out_type=jax.ShapeDtypeStruct((BATCH * OUT,), jnp.float32),
        compiler_params=pltpu.CompilerParams(use_tc_tiling_on_sc=False,
                                             needs_layout_passes=False),
        scratch_types=(
            [pltpu.VMEM((IDX_PER_W,), jnp.int32)]
            + [pltpu.VMEM((PAIR_IDX, OUT), jnp.float32) for _ in range(NBUF)]
            + [pltpu.VMEM((32,), jnp.float32),
               pltpu.VMEM((16,), jnp.float32),
               pltpu.VMEM((OUT_PER_W,), jnp.float32)]
            + [pltpu.SemaphoreType.DMA for _ in range(NBUF)]
        ),
    )
    def sc_kernel(idx_hbm, proj_hbm, bias2_hbm, out_hbm, idx_v,
                  buf0, buf1, buf2, buf3, fold_v, bias_v, out_v,
                  sem0, sem1, sem2, sem3):
        wid = lax.axis_index("s") * NUM_CORES + lax.axis_index("c")
        bufs = (buf0, buf1, buf2, buf3)
        sems = (sem0, sem1, sem2, sem3)

        idx_base = pl.multiple_of(wid * IDX_PER_W, 8)
        pltpu.sync_copy(idx_hbm.at[pl.ds(idx_base, IDX_PER_W)], idx_v)
        pltpu.sync_copy(bias2_hbm, bias_v)

        def fire(pair, buf, sem):
            off = pl.multiple_of(pair * PAIR_IDX, 8)
            coff = 0
            for c in CHUNKS:
                pltpu.async_copy(
                    proj_hbm.at[idx_v.at[pl.ds(off + coff, c)]],
                    buf.at[pl.ds(coff, c)], sem)
                coff += c

        def drain(buf, sem):
            coff = 0
            for c in CHUNKS:
                pltpu.make_async_copy(
                    proj_hbm.at[idx_v.at[pl.ds(0, c)]],
                    buf.at[pl.ds(coff, c)], sem).wait()
                coff += c

        for b in range(NBUF - 1):
            fire(b, bufs[b], sems[b])

        iota = lax.iota(jnp.int32, 16)
        evec = lax.shift_right_logical(iota, 3)
        cvec = lax.bitwise_and(iota, 7)
        fold_lo = cvec + evec * 16
        fold_hi = fold_lo + 8
        bias_vec = bias_v[...]

        def bag_sum(buf, rbase):
            def ld(rv):
                return plsc.load_gather(buf, [rv, cvec])

            rvs = [evec + (rbase + 2 * u) for u in range(8)]
            accs = [ld(rvs[u]) for u in range(8)]
            for _ in range(1, 12):
                rvs = [rv + 16 for rv in rvs]
                accs = [accs[u] + ld(rvs[u]) for u in range(8)]
            rvs = [rvs[u] + 16 for u in range(4)]
            accs = [accs[u] + ld(rvs[u]) if u < 4 else accs[u]
                    for u in range(8)]
            s01 = accs[0] + accs[1]
            s23 = accs[2] + accs[3]
            s45 = accs[4] + accs[5]
            s67 = accs[6] + accs[7]
            return (s01 + s23) + (s45 + s67)

        def quad_body(i, _):
            for p in range(NBUF):
                pair = i * NBUF + p
                nxt = (p + NBUF - 1) % NBUF

                @pl.when(pair + NBUF - 1 < PAIRS_PER_W)
                def _():
                    fire(pair + NBUF - 1, bufs[nxt], sems[nxt])

                drain(bufs[p], sems[p])
                fold_v[pl.ds(0, 16)] = bag_sum(bufs[p], 0)
                fold_v[pl.ds(16, 16)] = bag_sum(bufs[p], HIST)
                tot = (plsc.load_gather(fold_v, [fold_lo])
                       + plsc.load_gather(fold_v, [fold_hi])
                       + bias_vec)
                out_v[pl.ds(pl.multiple_of(pair * 16, 8), 16)] = tot
            return ()

        lax.fori_loop(0, PAIRS_PER_W // NBUF, quad_body, (), unroll=False)

        out_base = pl.multiple_of(wid * OUT_PER_W, 8)
        pltpu.sync_copy(out_v, out_hbm.at[pl.ds(out_base, OUT_PER_W)])

    return sc_kernel


def kernel(indices, offsets, emb_table, fc_w, fc_b):
    del offsets
    w_scaled = fc_w.T * jnp.float32(1.0 / HIST)
    proj = _tc_project(emb_table, w_scaled)
    bias2 = jnp.concatenate([fc_b, fc_b]).astype(jnp.float32)
    sc = _sc_bag_sum_kernel()
    return sc(indices, proj, bias2).reshape(BATCH, OUT)


# speedup vs baseline: 1.3087x; 1.3087x over previous
import functools

import jax
import jax.numpy as jnp
from jax import lax
from jax.experimental import pallas as pl
from jax.experimental.pallas import tpu as pltpu
from jax.experimental.pallas import tpu_sc as plsc

BATCH = 4096
HIST = 200
VOCAB = 100000
DIM = 16
OUT = 8
N = BATCH * HIST

NUM_CORES = 2
NUM_SUBCORES = 16
NUM_WORKERS = NUM_CORES * NUM_SUBCORES
BAGS_PER_W = BATCH // NUM_WORKERS
IDX_PER_W = BAGS_PER_W * HIST
OUT_PER_W = BAGS_PER_W * OUT

CHUNK_A = 128
CHUNK_B = HIST - CHUNK_A
NBUF = 4

PACK = 128 // DIM
PROJ_ROWS = VOCAB // PACK


def _tc_project(table2, big_w):
    def proj_kernel(t_ref, w_ref, o_ref):
        o_ref[...] = jnp.dot(t_ref[...], w_ref[...],
                             preferred_element_type=jnp.float32)

    return pl.pallas_call(
        proj_kernel,
        out_shape=jax.ShapeDtypeStruct((PROJ_ROWS, 128), jnp.float32),
    )(table2, big_w)


def _sc_bag_kernel():
    mesh = plsc.VectorSubcoreMesh(core_axis_name="c", subcore_axis_name="s")

    @functools.partial(
        pl.kernel,
        mesh=mesh,
        out_type=jax.ShapeDtypeStruct((BATCH * OUT,), jnp.float32),
        compiler_params=pltpu.CompilerParams(use_tc_tiling_on_sc=False,
                                             needs_layout_passes=False),
        scratch_types=(
            [pltpu.VMEM((IDX_PER_W,), jnp.int32)]
            + [pltpu.VMEM((HIST, DIM), jnp.float32) for _ in range(NBUF)]
            + [pltpu.VMEM((16,), jnp.float32),
               pltpu.VMEM((OUT_PER_W,), jnp.float32)]
            + [pltpu.SemaphoreType.DMA for _ in range(NBUF)]
        ),
    )
    def sc_kernel(idx_hbm, proj_hbm, bias2_hbm, out_hbm, idx_v,
                  buf0, buf1, buf2, buf3, bias_v, out_v,
                  sem0, sem1, sem2, sem3):
        wid = lax.axis_index("s") * NUM_CORES + lax.axis_index("c")
        bufs = (buf0, buf1, buf2, buf3)
        sems = (sem0, sem1, sem2, sem3)

        idx_base = pl.multiple_of(wid * IDX_PER_W, 8)
        pltpu.sync_copy(idx_hbm.at[pl.ds(idx_base, IDX_PER_W)], idx_v)
        pltpu.sync_copy(bias2_hbm, bias_v)

        def fire(bag, buf, sem):
            off = pl.multiple_of(bag * HIST, 8)
            pltpu.async_copy(
                proj_hbm.at[idx_v.at[pl.ds(off, CHUNK_A)]],
                buf.at[pl.ds(0, CHUNK_A)], sem)
            pltpu.async_copy(
                proj_hbm.at[idx_v.at[pl.ds(off + CHUNK_A, CHUNK_B)]],
                buf.at[pl.ds(CHUNK_A, CHUNK_B)], sem)

        def drain(buf, sem):
            pltpu.make_async_copy(
                proj_hbm.at[idx_v.at[pl.ds(0, CHUNK_A)]],
                buf.at[pl.ds(0, CHUNK_A)], sem).wait()
            pltpu.make_async_copy(
                proj_hbm.at[idx_v.at[pl.ds(0, CHUNK_B)]],
                buf.at[pl.ds(CHUNK_A, CHUNK_B)], sem).wait()

        for b in range(NBUF - 1):
            fire(b, bufs[b], sems[b])

        left_mask = lax.iota(jnp.int32, 16) < 8
        bias_vec = bias_v[...]

        def bag_sum(buf):
            accs = [buf[u] for u in range(8)]
            for j in range(1, HIST // 8):
                base = j * 8
                accs = [accs[u] + buf[base + u] for u in range(8)]
            s01 = accs[0] + accs[1]
            s23 = accs[2] + accs[3]
            s45 = accs[4] + accs[5]
            s67 = accs[6] + accs[7]
            return (s01 + s23) + (s45 + s67)

        def quad_body(i, _):
            acc_even = None
            for p in range(NBUF):
                bag = i * NBUF + p
                nxt = (p + NBUF - 1) % NBUF

                @pl.when(bag + NBUF - 1 < BAGS_PER_W)
                def _():
                    fire(bag + NBUF - 1, bufs[nxt], sems[nxt])

                drain(bufs[p], sems[p])
                acc = bag_sum(bufs[p])
                if p % 2 == 0:
                    acc_even = acc
                else:
                    tot = jnp.where(left_mask, acc_even, acc) + bias_vec
                    pair = i * 2 + p // 2
                    out_v[pl.ds(pl.multiple_of(pair * 16, 8), 16)] = tot
            return ()

        lax.fori_loop(0, BAGS_PER_W // NBUF, quad_body, (), unroll=False)

        out_base = pl.multiple_of(wid * OUT_PER_W, 8)
        pltpu.sync_copy(out_v, out_hbm.at[pl.ds(out_base, OUT_PER_W)])

    return sc_kernel


def kernel(indices, offsets, emb_table, fc_w, fc_b):
    del offsets
    w16 = jnp.concatenate([fc_w.T, fc_w.T], axis=1) * jnp.float32(1.0 / HIST)
    big_w = jnp.kron(jnp.eye(PACK, dtype=jnp.float32), w16)
    table2 = emb_table.reshape(PROJ_ROWS, 128)
    proj16 = _tc_project(table2, big_w).reshape(VOCAB, DIM)
    bias2 = jnp.concatenate([fc_b, fc_b]).astype(jnp.float32)
    sc = _sc_bag_kernel()
    return sc(indices, proj16, bias2).reshape(BATCH, OUT)
